# Initial kernel scaffold; baseline (speedup 1.0000x reference)
import jax
import jax.numpy as jnp
from jax.experimental import pallas as pl


def kernel(boxes, classification, detections):
    raise NotImplementedError("write your pallas kernel here")


# speedup vs baseline: 17.1390x; 17.1390x over previous
import jax
import jax.numpy as jnp
import numpy as np
from jax.experimental import pallas as pl

_NMS_THR = 0.4
_K = 1000
_MAX_OUT = 300
_R, _C = 160, 128
_P = _R * _C
_NEG = np.float32(-1e30)


def _nms_kernel(cls_ref, x1_ref, y1_ref, x2_ref, y2_ref, det_ref, out_ref):
    scores = jnp.max(cls_ref[...], axis=0)
    x1 = x1_ref[...]
    y1 = y1_ref[...]
    x2 = x2_ref[...]
    y2 = y2_ref[...]
    idx = (jax.lax.broadcasted_iota(jnp.int32, (_R, _C), 0) * _C
           + jax.lax.broadcasted_iota(jnp.int32, (_R, _C), 1))

    def bs_val(_, carry):
        lo, hi = carry
        mid = (lo + hi) // 2
        t = jax.lax.bitcast_convert_type(mid, jnp.float32)
        cnt = jnp.sum((scores >= t).astype(jnp.int32))
        big = cnt >= _K
        return jnp.where(big, mid, lo), jnp.where(big, hi, mid)

    lo0 = jnp.int32(0)
    hi0 = jnp.int32(0x3F800000)
    lo, hi = jax.lax.fori_loop(0, 31, bs_val, (lo0, hi0))
    vk = jax.lax.bitcast_convert_type(lo, jnp.float32)

    cnt_gt = jnp.sum((scores > vk).astype(jnp.int32))
    need = _K - cnt_gt
    eq = scores == vk

    def bs_idx(_, carry):
        lo2, hi2 = carry
        mid = (lo2 + hi2) // 2
        cnt = jnp.sum((eq & (idx <= mid)).astype(jnp.int32))
        ok = cnt >= need
        return jnp.where(ok, lo2, mid), jnp.where(ok, mid, hi2)

    lo2, hi2 = jax.lax.fori_loop(0, 16, bs_idx, (jnp.int32(-1), jnp.int32(_P - 1)))
    keep = (scores > vk) | (eq & (idx <= hi2))
    sw0 = jnp.where(keep, scores, _NEG)

    area_b = (jnp.maximum(x2 - x1, 0.0) * jnp.maximum(y2 - y1, 0.0))
    neg_half = _NEG / 2

    def nms_body(i, sw):
        m = jnp.max(sw)
        best = jnp.min(jnp.where(sw == m, idx, jnp.int32(_P)))
        is_valid = m > neg_half
        bx1 = jnp.sum(jnp.where(idx == best, x1, 0.0))
        by1 = jnp.sum(jnp.where(idx == best, y1, 0.0))
        bx2 = jnp.sum(jnp.where(idx == best, x2, 0.0))
        by2 = jnp.sum(jnp.where(idx == best, y2, 0.0))
        ix1 = jnp.maximum(bx1, x1)
        iy1 = jnp.maximum(by1, y1)
        ix2 = jnp.minimum(bx2, x2)
        iy2 = jnp.minimum(by2, y2)
        inter = jnp.maximum(ix2 - ix1, 0.0) * jnp.maximum(iy2 - iy1, 0.0)
        area_a = jnp.maximum(bx2 - bx1, 0.0) * jnp.maximum(by2 - by1, 0.0)
        union = area_a + area_b - inter
        iou = inter / jnp.maximum(union, 1e-9)
        suppress = (iou > _NMS_THR) | (idx == best)
        sw = jnp.where(is_valid & suppress, _NEG, sw)
        row = det_ref[pl.ds(best, 1), :]
        out_ref[pl.ds(i, 1), :] = jnp.where(is_valid, row, 0.0)
        return sw

    jax.lax.fori_loop(0, _MAX_OUT, nms_body, sw0)


def kernel(boxes, classification, detections):
    b = boxes[0]
    cls = classification[0]
    det = detections[0]
    n = b.shape[0]
    pad = _P - n
    clsp = jnp.pad(cls, ((0, pad), (0, 0)), constant_values=-1e30)
    cls_t = clsp.T.reshape(cls.shape[1], _R, _C)
    bp = jnp.pad(b, ((0, pad), (0, 0)))
    x1 = bp[:, 0].reshape(_R, _C)
    y1 = bp[:, 1].reshape(_R, _C)
    x2 = bp[:, 2].reshape(_R, _C)
    y2 = bp[:, 3].reshape(_R, _C)
    detp = jnp.pad(det, ((0, pad), (0, 0)))
    out = pl.pallas_call(
        _nms_kernel,
        out_shape=jax.ShapeDtypeStruct((_MAX_OUT, det.shape[1]), jnp.float32),
    )(cls_t, x1, y1, x2, y2, detp)
    return out[None]
